# baseline (speedup 1.0000x reference)
import functools

import jax
import jax.numpy as jnp
from jax import lax
from jax.experimental import pallas as pl
from jax.experimental.pallas import tpu as pltpu
from jax.experimental.pallas import tpu_sc as plsc

N = 10000
E = 320000
D = 128
K = 2500
KP = 2560
SW = 128
SKT = KP // SW
BI = 400
NBI = N // BI
NW = 32
EB = 112
CAP = 11200
ACC_R = 328
TRASH = 324
CT = 640
NCT = KP // CT


def _assoc_body(h_ref, xrep_ref, s2d_ref, st_ref, out_ref):
    i = pl.program_id(0)
    hb = h_ref[...]
    xr = xrep_ref[...]
    logits = lax.dot_general(hb, xr, (((1,), (1,)), ((), ())),
                             preferred_element_type=jnp.float32)
    colmask = lax.broadcasted_iota(jnp.int32, (BI, KP), 1) < K
    lm = jnp.where(colmask, logits, -jnp.inf)
    m = jnp.max(lm, axis=1, keepdims=True)
    e = jnp.where(colmask, jnp.exp(logits - m), 0.0)
    z = jnp.sum(e, axis=1, keepdims=True)
    p = e / z
    s2d_ref[...] = p
    for kt in range(SKT):
        st_ref[kt] = p[:, kt * SW:(kt + 1) * SW]

    @pl.when(i == 0)
    def _():
        out_ref[...] = jnp.zeros_like(out_ref)

    out_ref[...] += lax.dot_general(p, hb, (((0,), (0,)), ((), ())),
                                    preferred_element_type=jnp.float32)


def _assoc(h, xrep_pad):
    return pl.pallas_call(
        _assoc_body,
        grid=(NBI,),
        in_specs=[
            pl.BlockSpec((BI, D), lambda i: (i, 0)),
            pl.BlockSpec((KP, D), lambda i: (0, 0)),
        ],
        out_specs=[
            pl.BlockSpec((BI, KP), lambda i: (i, 0)),
            pl.BlockSpec((SKT, BI, SW), lambda i: (0, i, 0)),
            pl.BlockSpec((KP, D), lambda i: (0, 0)),
        ],
        out_shape=[
            jax.ShapeDtypeStruct((N, KP), jnp.float32),
            jax.ShapeDtypeStruct((SKT, N, SW), jnp.float32),
            jax.ShapeDtypeStruct((KP, D), jnp.float32),
        ],
    )(h, xrep_pad)


def _t_kernel_body(sflat, srclists, dstlists, zeros_hbm, out_hbm,
                   srclist, dstlist, idx0, idx1,
                   rows0, rows1, acc, semg0, semg1):
    c = lax.axis_index("c")
    s = lax.axis_index("s")
    w = c * 16 + s
    lo = ((N * w) // NW) // 8 * 8
    hi = ((N * (w + 1)) // NW) // 8 * 8
    rng = hi - lo

    pltpu.sync_copy(srclists.at[w], srclist)
    pltpu.sync_copy(dstlists.at[w], dstlist)
    nb = CAP // EB

    bufs = ((idx0, rows0, semg0), (idx1, rows1, semg1))

    def issue_gather(kt, b, j):
        idxb, rows, semg = bufs[j]
        off = kt * N

        def add_off(u, _):
            idxb[pl.ds(u * 16, 16)] = (
                srclist[pl.ds(b * EB + u * 16, 16)] + off)
            return 0

        lax.fori_loop(0, EB // 16, add_off, 0)
        pltpu.async_copy(sflat.at[idxb], rows, semg)

    def kt_body(kt, _):
        pltpu.sync_copy(zeros_hbm, acc)

        @pl.when(nb > 0)
        def _():
            issue_gather(kt, 0, 0)

        def pair_body(i, _):
            for j in (0, 1):
                b = 2 * i + j
                idxb, rows, semg = bufs[j]

                @pl.when(b < nb)
                def _():
                    pltpu.make_async_copy(sflat.at[idxb], rows, semg).wait()

                    @pl.when(b + 1 < nb)
                    def _():
                        issue_gather(kt, b + 1, 1 - j)

                    def acc_body(g, _):
                        dv = dstlist[pl.ds(b * EB + g * 16, 16)]
                        for t in range(16):
                            d = dv[t]
                            e = g * 16 + t
                            for u in range(SW // 16):
                                sl = pl.ds(u * 16, 16)
                                acc[d, sl] += rows[e, sl]
                        return 0

                    lax.fori_loop(0, EB // 16, acc_body, 0)
            return 0

        lax.fori_loop(0, (nb + 1) // 2, pair_body, 0)

        @pl.when(rng == 312)
        def _():
            pltpu.sync_copy(acc.at[pl.ds(0, 312)],
                            out_hbm.at[kt, pl.ds(lo, 312)])

        @pl.when(rng == 320)
        def _():
            pltpu.sync_copy(acc.at[pl.ds(0, 320)],
                            out_hbm.at[kt, pl.ds(lo, 320)])

        return 0

    lax.fori_loop(0, SKT, kt_body, 0)


def _segment_sum_sc(s_flat, srclists, dstlists, zeros_pad):
    mesh = plsc.VectorSubcoreMesh(core_axis_name="c", subcore_axis_name="s")
    fn = functools.partial(
        pl.kernel,
        out_type=jax.ShapeDtypeStruct((SKT, N, SW), jnp.float32),
        mesh=mesh,
        scratch_types=[
            pltpu.VMEM((CAP,), jnp.int32),
            pltpu.VMEM((CAP,), jnp.int32),
            pltpu.VMEM((EB,), jnp.int32),
            pltpu.VMEM((EB,), jnp.int32),
            pltpu.VMEM((EB, SW), jnp.float32),
            pltpu.VMEM((EB, SW), jnp.float32),
            pltpu.VMEM((ACC_R, SW), jnp.float32),
            pltpu.SemaphoreType.DMA,
            pltpu.SemaphoreType.DMA,
        ],
    )(_t_kernel_body)
    return fn(s_flat, srclists, dstlists, zeros_pad)


def _adj_body(s2d_ref, t_ref, adj_ref):
    i = pl.program_id(1)

    @pl.when(i == 0)
    def _():
        adj_ref[...] = jnp.zeros_like(adj_ref)

    sb = s2d_ref[...].astype(jnp.bfloat16)
    for j in range(CT // SW):
        tj = t_ref[j].astype(jnp.bfloat16)
        adj_ref[:, j * SW:(j + 1) * SW] += lax.dot_general(
            sb, tj, (((0,), (0,)), ((), ())),
            preferred_element_type=jnp.float32)


def _adj(s2d, tpart):
    return pl.pallas_call(
        _adj_body,
        grid=(NCT, NBI),
        in_specs=[
            pl.BlockSpec((BI, KP), lambda ct, i: (i, 0)),
            pl.BlockSpec((CT // SW, BI, SW), lambda ct, i: (ct, i, 0)),
        ],
        out_specs=pl.BlockSpec((KP, CT), lambda ct, i: (0, ct)),
        out_shape=jax.ShapeDtypeStruct((KP, KP), jnp.float32),
    )(s2d, tpart)


def kernel(x, edge_index, gcn_weight, gcn_bias, select_weight):
    src, dst = edge_index[0], edge_index[1]

    loop = jnp.arange(N, dtype=edge_index.dtype)
    src2 = jnp.concatenate([src, loop])
    dst2 = jnp.concatenate([dst, loop])
    deg = jnp.zeros((N,), jnp.float32).at[dst2].add(1.0)
    dinv = jax.lax.rsqrt(jnp.maximum(deg, 1e-12))
    norm = dinv[src2] * dinv[dst2]
    h = x @ gcn_weight
    h = jax.ops.segment_sum(h[src2] * norm[:, None], dst2,
                            num_segments=N) + gcn_bias
    score = jnp.tanh(h @ select_weight / jnp.linalg.norm(select_weight))
    vals, perm = jax.lax.top_k(score, K)
    xrep = h[perm] * vals[:, None]
    xrep_pad = jnp.zeros((KP, D), jnp.float32).at[:K].set(xrep)

    s2d, s_tiled, out_pad = _assoc(h, xrep_pad)

    s_flat = s_tiled.reshape(SKT * N, SW)
    zeros_pad = jnp.zeros((ACC_R, SW), jnp.float32)
    los = jnp.array([((N * wq) // NW) // 8 * 8 for wq in range(NW + 1)],
                    jnp.int32)
    w_est = (dst * NW) // N
    w_e = (w_est
           + (dst >= los[w_est + 1]).astype(jnp.int32)
           - (dst < los[w_est]).astype(jnp.int32))
    order = jnp.argsort(w_e)
    w_s = w_e[order]
    src_s = src[order]
    dstl_s = dst[order] - los[w_s]
    starts = jnp.searchsorted(w_s, jnp.arange(NW, dtype=jnp.int32))
    posn = jnp.arange(E, dtype=jnp.int32) - starts[w_s].astype(jnp.int32)
    tgt = jnp.where(posn < CAP, w_s * CAP + posn, NW * CAP)
    srclists = jnp.zeros((NW * CAP + 8,), jnp.int32).at[tgt].set(src_s)
    dstlists = jnp.full((NW * CAP + 8,), TRASH, jnp.int32).at[tgt].set(dstl_s)
    srclists = srclists[:NW * CAP].reshape(NW, CAP)
    dstlists = dstlists[:NW * CAP].reshape(NW, CAP)
    tpart = _segment_sum_sc(s_flat, srclists, dstlists, zeros_pad)

    adj_pad = _adj(s2d, tpart)

    out = out_pad[:K]
    adj_pooled = adj_pad[:K, :K]
    S = s2d[:, :K]
    return out, adj_pooled, S


# speedup vs baseline: 2.0164x; 2.0164x over previous
import functools

import jax
import jax.numpy as jnp
from jax import lax
from jax.experimental import pallas as pl
from jax.experimental.pallas import tpu as pltpu
from jax.experimental.pallas import tpu_sc as plsc

N = 10000
E = 320000
D = 128
K = 2500
KP = 2560
NKT = KP // 128
BI = 400
NBI = N // BI
NW = 32
EPW = E // NW
EB = 80
NB_E = EPW // EB
NACC = 10240
RPT = NACC // 16
CT = 640
NCT = KP // CT


def _assoc_body(h_ref, xrep_ref, s2d_ref, st_ref, out_ref):
    i = pl.program_id(0)
    hb = h_ref[...]
    xr = xrep_ref[...]
    logits = lax.dot_general(hb, xr, (((1,), (1,)), ((), ())),
                             preferred_element_type=jnp.float32)
    colmask = lax.broadcasted_iota(jnp.int32, (BI, KP), 1) < K
    lm = jnp.where(colmask, logits, -jnp.inf)
    m = jnp.max(lm, axis=1, keepdims=True)
    e = jnp.where(colmask, jnp.exp(logits - m), 0.0)
    z = jnp.sum(e, axis=1, keepdims=True)
    p = e / z
    s2d_ref[...] = p
    for kt in range(NKT):
        st_ref[kt] = p[:, kt * 128:(kt + 1) * 128]

    @pl.when(i == 0)
    def _():
        out_ref[...] = jnp.zeros_like(out_ref)

    out_ref[...] += lax.dot_general(p, hb, (((0,), (0,)), ((), ())),
                                    preferred_element_type=jnp.float32)


def _assoc(h, xrep_pad):
    return pl.pallas_call(
        _assoc_body,
        grid=(NBI,),
        in_specs=[
            pl.BlockSpec((BI, D), lambda i: (i, 0)),
            pl.BlockSpec((KP, D), lambda i: (0, 0)),
        ],
        out_specs=[
            pl.BlockSpec((BI, KP), lambda i: (i, 0)),
            pl.BlockSpec((NKT, BI, 128), lambda i: (0, i, 0)),
            pl.BlockSpec((KP, D), lambda i: (0, 0)),
        ],
        out_shape=[
            jax.ShapeDtypeStruct((N, KP), jnp.float32),
            jax.ShapeDtypeStruct((NKT, N, 128), jnp.float32),
            jax.ShapeDtypeStruct((KP, D), jnp.float32),
        ],
    )(h, xrep_pad)


def _t_kernel_body(sflat, src2d, dst2d, zeros_hbm, out_hbm,
                   srcb, dstb, idxb, rows, acc, sem):
    c = lax.axis_index("c")
    s = lax.axis_index("s")
    w = c * 16 + s

    def kt_body(kt, _):
        pltpu.sync_copy(zeros_hbm, acc.at[pl.ds(s * RPT, RPT)])
        plsc.subcore_barrier()

        def eb_body(b, _):
            row = w * NB_E + b
            pltpu.sync_copy(src2d.at[pl.ds(row, 1)], srcb)
            pltpu.sync_copy(dst2d.at[pl.ds(row, 1)], dstb)
            off = kt * N

            def add_off(j, _):
                idxb[pl.ds(j * 16, 16)] = srcb[0, pl.ds(j * 16, 16)] + off
                return 0

            lax.fori_loop(0, EB // 16, add_off, 0)
            pltpu.async_copy(sflat.at[idxb], rows, sem).wait()
            pltpu.sync_copy(rows, acc.at[dstb.at[0]], add=True)
            return 0

        lax.fori_loop(0, NB_E, eb_body, 0)
        plsc.subcore_barrier()

        @pl.when(s < 15)
        def _():
            pltpu.sync_copy(acc.at[pl.ds(s * RPT, RPT)],
                            out_hbm.at[c, kt, pl.ds(s * RPT, RPT)])

        @pl.when(s == 15)
        def _():
            pltpu.sync_copy(acc.at[pl.ds(15 * RPT, N - 15 * RPT)],
                            out_hbm.at[c, kt, pl.ds(15 * RPT, N - 15 * RPT)])

        plsc.subcore_barrier()
        return 0

    lax.fori_loop(0, NKT, kt_body, 0)


def _segment_sum_sc(s_flat, src2d, dst2d, zeros_pad):
    mesh = plsc.VectorSubcoreMesh(core_axis_name="c", subcore_axis_name="s")
    fn = functools.partial(
        pl.kernel,
        out_type=jax.ShapeDtypeStruct((2, NKT, N, 128), jnp.float32),
        mesh=mesh,
        scratch_types=[
            pltpu.VMEM((1, EB), jnp.int32),
            pltpu.VMEM((1, EB), jnp.int32),
            pltpu.VMEM((EB,), jnp.int32),
            pltpu.VMEM((EB, 128), jnp.float32),
            pltpu.VMEM_SHARED((NACC, 128), jnp.float32),
            pltpu.SemaphoreType.DMA,
        ],
    )(_t_kernel_body)
    return fn(s_flat, src2d, dst2d, zeros_pad)


def _adj_body(s2d_ref, t0_ref, t1_ref, adj_ref):
    i = pl.program_id(1)

    @pl.when(i == 0)
    def _():
        adj_ref[...] = jnp.zeros_like(adj_ref)

    sb = s2d_ref[...].astype(jnp.bfloat16)
    for j in range(CT // 128):
        tj = (t0_ref[0, j] + t1_ref[0, j]).astype(jnp.bfloat16)
        adj_ref[:, j * 128:(j + 1) * 128] += lax.dot_general(
            sb, tj, (((0,), (0,)), ((), ())),
            preferred_element_type=jnp.float32)


def _adj(s2d, tpart):
    return pl.pallas_call(
        _adj_body,
        grid=(NCT, NBI),
        in_specs=[
            pl.BlockSpec((BI, KP), lambda ct, i: (i, 0)),
            pl.BlockSpec((1, CT // 128, BI, 128), lambda ct, i: (0, ct, i, 0)),
            pl.BlockSpec((1, CT // 128, BI, 128), lambda ct, i: (1, ct, i, 0)),
        ],
        out_specs=pl.BlockSpec((KP, CT), lambda ct, i: (0, ct)),
        out_shape=jax.ShapeDtypeStruct((KP, KP), jnp.float32),
    )(s2d, tpart, tpart)


def kernel(x, edge_index, gcn_weight, gcn_bias, select_weight):
    src, dst = edge_index[0], edge_index[1]

    loop = jnp.arange(N, dtype=edge_index.dtype)
    src2 = jnp.concatenate([src, loop])
    dst2 = jnp.concatenate([dst, loop])
    deg = jnp.zeros((N,), jnp.float32).at[dst2].add(1.0)
    dinv = jax.lax.rsqrt(jnp.maximum(deg, 1e-12))
    norm = dinv[src2] * dinv[dst2]
    h = x @ gcn_weight
    h = jax.ops.segment_sum(h[src2] * norm[:, None], dst2,
                            num_segments=N) + gcn_bias
    score = jnp.tanh(h @ select_weight / jnp.linalg.norm(select_weight))
    vals, perm = jax.lax.top_k(score, K)
    xrep = h[perm] * vals[:, None]
    xrep_pad = jnp.zeros((KP, D), jnp.float32).at[:K].set(xrep)

    s2d, s_tiled, out_pad = _assoc(h, xrep_pad)

    s_flat = s_tiled.reshape(NKT * N, 128)
    src2d = src.reshape(NW * NB_E, EB)
    dst2d = dst.reshape(NW * NB_E, EB)
    zeros_pad = jnp.zeros((RPT, 128), jnp.float32)
    tpart = _segment_sum_sc(s_flat, src2d, dst2d, zeros_pad)

    adj_pad = _adj(s2d, tpart)

    out = out_pad[:K]
    adj_pooled = adj_pad[:K, :K]
    S = s2d[:, :K]
    return out, adj_pooled, S


# speedup vs baseline: 2.6324x; 1.3055x over previous
import functools

import jax
import jax.numpy as jnp
from jax import lax
from jax.experimental import pallas as pl
from jax.experimental.pallas import tpu as pltpu
from jax.experimental.pallas import tpu_sc as plsc

N = 10000
E = 320000
D = 128
K = 2500
KP = 2560
NKT = KP // 128
BI = 400
NBI = N // BI
NW = 32
EPW = E // NW
EB = 80
NB_E = EPW // EB
NACC = 10240
RPT = NACC // 16
CT = 640
NCT = KP // CT


def _assoc_body(h_ref, xrep_ref, s2d_ref, st_ref, out_ref):
    i = pl.program_id(0)
    hb = h_ref[...]
    xr = xrep_ref[...]
    logits = lax.dot_general(hb, xr, (((1,), (1,)), ((), ())),
                             preferred_element_type=jnp.float32)
    colmask = lax.broadcasted_iota(jnp.int32, (BI, KP), 1) < K
    lm = jnp.where(colmask, logits, -jnp.inf)
    m = jnp.max(lm, axis=1, keepdims=True)
    e = jnp.where(colmask, jnp.exp(logits - m), 0.0)
    z = jnp.sum(e, axis=1, keepdims=True)
    p = e / z
    s2d_ref[...] = p
    for kt in range(NKT):
        st_ref[kt] = p[:, kt * 128:(kt + 1) * 128]

    @pl.when(i == 0)
    def _():
        out_ref[...] = jnp.zeros_like(out_ref)

    out_ref[...] += lax.dot_general(p, hb, (((0,), (0,)), ((), ())),
                                    preferred_element_type=jnp.float32)


def _assoc(h, xrep_pad):
    return pl.pallas_call(
        _assoc_body,
        grid=(NBI,),
        in_specs=[
            pl.BlockSpec((BI, D), lambda i: (i, 0)),
            pl.BlockSpec((KP, D), lambda i: (0, 0)),
        ],
        out_specs=[
            pl.BlockSpec((BI, KP), lambda i: (i, 0)),
            pl.BlockSpec((NKT, BI, 128), lambda i: (0, i, 0)),
            pl.BlockSpec((KP, D), lambda i: (0, 0)),
        ],
        out_shape=[
            jax.ShapeDtypeStruct((N, KP), jnp.float32),
            jax.ShapeDtypeStruct((NKT, N, 128), jnp.float32),
            jax.ShapeDtypeStruct((KP, D), jnp.float32),
        ],
    )(h, xrep_pad)


def _t_kernel_body(sflat, src_hbm, dst_hbm, zeros_hbm, out_hbm,
                   srclist, dstlist, idx0, idx1, rows0, rows1, acc,
                   sem0, sem1):
    c = lax.axis_index("c")
    s = lax.axis_index("s")
    w = c * 16 + s

    pltpu.sync_copy(src_hbm.at[w], srclist)
    pltpu.sync_copy(dst_hbm.at[w], dstlist)

    bufs = ((idx0, rows0, sem0), (idx1, rows1, sem1))

    def issue_gather(kt, b, j):
        idxb, rows, sem = bufs[j]
        off = kt * N

        def add_off(u, _):
            idxb[pl.ds(u * 16, 16)] = srclist[pl.ds(b * EB + u * 16, 16)] + off
            return 0

        lax.fori_loop(0, EB // 16, add_off, 0)
        pltpu.async_copy(sflat.at[idxb], rows, sem)

    def kt_body(kt, _):
        pltpu.sync_copy(zeros_hbm, acc.at[pl.ds(s * RPT, RPT)])
        plsc.subcore_barrier()
        issue_gather(kt, 0, 0)

        def pair_body(i, _):
            for j in (0, 1):
                b = 2 * i + j
                idxb, rows, sem = bufs[j]

                @pl.when(b < NB_E)
                def _():
                    pltpu.make_async_copy(sflat.at[idxb], rows, sem).wait()

                    @pl.when(b + 1 < NB_E)
                    def _():
                        issue_gather(kt, b + 1, 1 - j)

                    pltpu.sync_copy(rows, acc.at[dstlist.at[b]], add=True)
            return 0

        lax.fori_loop(0, (NB_E + 1) // 2, pair_body, 0)
        plsc.subcore_barrier()

        @pl.when(s < 15)
        def _():
            pltpu.sync_copy(acc.at[pl.ds(s * RPT, RPT)],
                            out_hbm.at[c, kt, pl.ds(s * RPT, RPT)])

        @pl.when(s == 15)
        def _():
            pltpu.sync_copy(acc.at[pl.ds(15 * RPT, N - 15 * RPT)],
                            out_hbm.at[c, kt, pl.ds(15 * RPT, N - 15 * RPT)])

        plsc.subcore_barrier()
        return 0

    lax.fori_loop(0, NKT, kt_body, 0)


def _segment_sum_sc(s_flat, src2d, dst2d, zeros_pad):
    mesh = plsc.VectorSubcoreMesh(core_axis_name="c", subcore_axis_name="s")
    fn = functools.partial(
        pl.kernel,
        out_type=jax.ShapeDtypeStruct((2, NKT, N, 128), jnp.float32),
        mesh=mesh,
        scratch_types=[
            pltpu.VMEM((EPW,), jnp.int32),
            pltpu.VMEM((NB_E, EB), jnp.int32),
            pltpu.VMEM((EB,), jnp.int32),
            pltpu.VMEM((EB,), jnp.int32),
            pltpu.VMEM((EB, 128), jnp.float32),
            pltpu.VMEM((EB, 128), jnp.float32),
            pltpu.VMEM_SHARED((NACC, 128), jnp.float32),
            pltpu.SemaphoreType.DMA,
            pltpu.SemaphoreType.DMA,
        ],
    )(_t_kernel_body)
    return fn(s_flat, src2d, dst2d, zeros_pad)


def _adj_body(s2d_ref, t0_ref, t1_ref, adj_ref):
    i = pl.program_id(1)

    @pl.when(i == 0)
    def _():
        adj_ref[...] = jnp.zeros_like(adj_ref)

    sb = s2d_ref[...].astype(jnp.bfloat16)
    for j in range(CT // 128):
        tj = (t0_ref[0, j] + t1_ref[0, j]).astype(jnp.bfloat16)
        adj_ref[:, j * 128:(j + 1) * 128] += lax.dot_general(
            sb, tj, (((0,), (0,)), ((), ())),
            preferred_element_type=jnp.float32)


def _adj(s2d, tpart):
    return pl.pallas_call(
        _adj_body,
        grid=(NCT, NBI),
        in_specs=[
            pl.BlockSpec((BI, KP), lambda ct, i: (i, 0)),
            pl.BlockSpec((1, CT // 128, BI, 128), lambda ct, i: (0, ct, i, 0)),
            pl.BlockSpec((1, CT // 128, BI, 128), lambda ct, i: (1, ct, i, 0)),
        ],
        out_specs=pl.BlockSpec((KP, CT), lambda ct, i: (0, ct)),
        out_shape=jax.ShapeDtypeStruct((KP, KP), jnp.float32),
    )(s2d, tpart, tpart)


def kernel(x, edge_index, gcn_weight, gcn_bias, select_weight):
    src, dst = edge_index[0], edge_index[1]

    loop = jnp.arange(N, dtype=edge_index.dtype)
    src2 = jnp.concatenate([src, loop])
    dst2 = jnp.concatenate([dst, loop])
    deg = jnp.zeros((N,), jnp.float32).at[dst2].add(1.0)
    dinv = jax.lax.rsqrt(jnp.maximum(deg, 1e-12))
    norm = dinv[src2] * dinv[dst2]
    h = x @ gcn_weight
    h = jax.ops.segment_sum(h[src2] * norm[:, None], dst2,
                            num_segments=N) + gcn_bias
    score = jnp.tanh(h @ select_weight / jnp.linalg.norm(select_weight))
    vals, perm = jax.lax.top_k(score, K)
    xrep = h[perm] * vals[:, None]
    xrep_pad = jnp.zeros((KP, D), jnp.float32).at[:K].set(xrep)

    s2d, s_tiled, out_pad = _assoc(h, xrep_pad)

    s_flat = s_tiled.reshape(NKT * N, 128)
    src2d = src.reshape(NW, EPW)
    dst2d = dst.reshape(NW, NB_E, EB)
    zeros_pad = jnp.zeros((RPT, 128), jnp.float32)
    tpart = _segment_sum_sc(s_flat, src2d, dst2d, zeros_pad)

    adj_pad = _adj(s2d, tpart)

    out = out_pad[:K]
    adj_pooled = adj_pad[:K, :K]
    S = s2d[:, :K]
    return out, adj_pooled, S
